# baseline (speedup 1.0000x reference)
import functools

import jax
import jax.numpy as jnp
import numpy as np
from jax import lax
from jax.experimental import pallas as pl
from jax.experimental.pallas import tpu as pltpu
from jax.experimental.pallas import tpu_sc as plsc

_BINS = 10
_EDGES_F32 = (np.arange(1, _BINS, dtype=np.float32) / np.float32(_BINS))
_TAUS = np.log(_EDGES_F32.astype(np.float64)
               / (1.0 - _EDGES_F32.astype(np.float64))).astype(np.float32)

_NEDGE = _BINS - 1
_NACC = 2 * _NEDGE + 1


def _splat(v, dtype=jnp.float32):
    return jnp.full((16,), v, dtype=dtype)


def _sc_partials(d_arr, target, start, count):
    info = plsc.get_sparse_core_info()
    ncores, nsub = info.num_cores, info.num_subcores
    nworkers = ncores * nsub
    assert count % (nworkers * 16) == 0
    per_worker = count // nworkers
    chunk = 8192 if per_worker % 8192 == 0 else per_worker
    nchunks = per_worker // chunk
    nvec = chunk // 16

    mesh = plsc.VectorSubcoreMesh(core_axis_name="c", subcore_axis_name="s")

    @functools.partial(
        pl.kernel,
        mesh=mesh,
        compiler_params=pltpu.CompilerParams(needs_layout_passes=False),
        out_type=jax.ShapeDtypeStruct((nworkers, _NACC, 16), jnp.float32),
        scratch_types=[
            pltpu.VMEM((chunk,), jnp.float32),
            pltpu.VMEM((chunk,), jnp.float32),
            pltpu.VMEM((chunk,), jnp.int32),
            pltpu.VMEM((chunk,), jnp.int32),
            pltpu.VMEM((_NACC, 16), jnp.float32),
            pltpu.SemaphoreType.DMA,
            pltpu.SemaphoreType.DMA,
            pltpu.SemaphoreType.DMA,
            pltpu.SemaphoreType.DMA,
        ],
    )
    def sc_kernel(in_hbm, tgt_hbm, out_hbm, in_buf0, in_buf1, tgt_buf0,
                  tgt_buf1, acc_v, sem_i0, sem_i1, sem_t0, sem_t1):
        in_bufs = (in_buf0, in_buf1)
        tgt_bufs = (tgt_buf0, tgt_buf1)
        wid = lax.axis_index("s") * ncores + lax.axis_index("c")
        base = start + wid * per_worker
        sems_i = (sem_i0, sem_i1)
        sems_t = (sem_t0, sem_t1)

        def launch(c):
            slot = c % 2
            off = base + c * chunk
            cp_i = pltpu.async_copy(
                in_hbm.at[pl.ds(off, chunk)], in_bufs[slot], sems_i[slot])
            cp_t = pltpu.async_copy(
                tgt_hbm.at[pl.ds(off, chunk)], tgt_bufs[slot], sems_t[slot])
            return cp_i, cp_t

        ones_i = _splat(1, jnp.int32)
        zero = _splat(0.0)
        pcoef = [_splat(v) for v in (
            2.554673e-07, 0.9999671, -0.49928504, 0.32722571, -0.22316587,
            0.13083343, -0.052437536, 0.01000929)]
        taus = [_splat(float(t)) for t in _TAUS]

        zero_cnt = _splat(0, jnp.int32)
        accs = [zero_cnt] * _NEDGE + [zero] * (_NEDGE + 1)

        pending = launch(0)
        for c in range(nchunks):
            nxt = launch(c + 1) if c + 1 < nchunks else None
            pending[0].wait()
            pending[1].wait()
            slot = c % 2
            in_view = in_bufs[slot]
            tgt_view = tgt_bufs[slot]

            def body(v, carry):
                d = in_view[pl.ds(v * 16, 16)]
                t = tgt_view[pl.ds(v * 16, 16)]
                u = jnp.where(t == ones_i, d, -d)
                e = jnp.exp(-jnp.abs(d))
                log1pe = pcoef[0] + e * (pcoef[1] + e * (pcoef[2] + e * (
                    pcoef[3] + e * (pcoef[4] + e * (pcoef[5] + e * (
                        pcoef[6] + e * pcoef[7]))))))
                ce = jnp.maximum(u, zero) + log1pe
                out = list(carry)
                out[2 * _NEDGE] = out[2 * _NEDGE] + ce
                for i in range(_NEDGE):
                    m = d >= taus[i]
                    out[i] = out[i] + plsc.all_reduce_population_count(m)
                    out[_NEDGE + i] = out[_NEDGE + i] + jnp.where(m, ce, zero)
                return tuple(out)

            accs = list(lax.fori_loop(0, nvec, body, tuple(accs)))
            pending = nxt

        for i in range(_NACC):
            acc_v[i] = accs[i].astype(jnp.float32)
        pltpu.sync_copy(acc_v, out_hbm.at[wid])

    return sc_kernel(d_arr, target)


def _tc_hist_body(d_ref, t_ref, out_ref):
    @pl.when(pl.program_id(0) == 0)
    def _():
        out_ref[...] = jnp.zeros_like(out_ref)

    d = d_ref[...]
    t = t_ref[...]
    u = jnp.where(t == 1, d, -d)
    e = jnp.exp(-jnp.abs(d))
    ce = jnp.maximum(u, 0.0) + jnp.log1p(e)
    acc = out_ref[...]
    rows = [None] * _NACC
    rows[2 * _NEDGE] = jnp.sum(ce, axis=0)
    for i in range(_NEDGE):
        m = d >= _TAUS[i]
        rows[i] = jnp.sum(jnp.where(m, 1.0, 0.0), axis=0)
        rows[_NEDGE + i] = jnp.sum(jnp.where(m, ce, 0.0), axis=0)
    out_ref[...] = acc + jnp.stack(rows, axis=0)


def _tc_partials(d2, t2):
    rows = d2.shape[0]
    blk = 512
    assert rows % blk == 0
    return pl.pallas_call(
        _tc_hist_body,
        grid=(rows // blk,),
        in_specs=[
            pl.BlockSpec((blk, 128), lambda i: (i, 0)),
            pl.BlockSpec((blk, 128), lambda i: (i, 0)),
        ],
        out_specs=pl.BlockSpec((_NACC, 128), lambda i: (0, 0)),
        out_shape=jax.ShapeDtypeStruct((_NACC, 128), jnp.float32),
        compiler_params=pltpu.CompilerParams(
            dimension_semantics=("arbitrary",)),
    )(d2, t2)


def _finalize_body(n, sc_ref, tc_ref, out_ref):
    x = sc_ref[...]
    s2 = jnp.sum(x, axis=0)
    sc_rows = jnp.sum(s2, axis=1)
    tc_rows = jnp.sum(tc_ref[...], axis=1)
    cnt_cum = (sc_rows[0:_NEDGE] * np.float32(1.0 / 16.0)
               + tc_rows[0:_NEDGE])
    ce_cum = sc_rows[_NEDGE:2 * _NEDGE] + tc_rows[_NEDGE:2 * _NEDGE]
    ce_tot = sc_rows[2 * _NEDGE] + tc_rows[2 * _NEDGE]
    n_f = jnp.full((1,), float(n), jnp.float32)
    zero1 = jnp.zeros((1,), jnp.float32)
    s_lo = jnp.concatenate([n_f, cnt_cum])
    s_hi = jnp.concatenate([cnt_cum, zero1])
    ce_lo = jnp.concatenate([jnp.reshape(ce_tot, (1,)), ce_cum])
    ce_hi = jnp.concatenate([ce_cum, zero1])
    cnt_b = s_lo - s_hi
    ce_b = ce_lo - ce_hi
    per_bin = jnp.where(cnt_b > 0.5, ce_b / jnp.maximum(cnt_b, 1.0), 0.0)
    loss = jnp.sum(per_bin) * np.float32(n / _BINS)
    out_ref[...] = jnp.reshape(loss, (1, 1))


_SC_SHARE_NUM, _SC_SHARE_DEN = 3, 8


def kernel(inputs, target):
    n = inputs.shape[0]
    target = target.astype(jnp.int32)
    d_arr = inputs[:, 0] - inputs[:, 1]
    n_sc = (n * _SC_SHARE_NUM // _SC_SHARE_DEN) // 262144 * 262144
    n_tc = n - n_sc
    part_sc = _sc_partials(d_arr, target, n_tc, n_sc)
    part_tc = _tc_partials(d_arr[:n_tc].reshape(-1, 128),
                           target[:n_tc].reshape(-1, 128))
    loss = pl.pallas_call(
        functools.partial(_finalize_body, n),
        out_shape=jax.ShapeDtypeStruct((1, 1), jnp.float32),
    )(part_sc, part_tc)
    return jnp.reshape(loss, ())


# speedup vs baseline: 1.0614x; 1.0614x over previous
import functools

import jax
import jax.numpy as jnp
import numpy as np
from jax import lax
from jax.experimental import pallas as pl
from jax.experimental.pallas import tpu as pltpu
from jax.experimental.pallas import tpu_sc as plsc

_BINS = 10
_EDGES_F32 = (np.arange(1, _BINS, dtype=np.float32) / np.float32(_BINS))
_TAUS = np.log(_EDGES_F32.astype(np.float64)
               / (1.0 - _EDGES_F32.astype(np.float64))).astype(np.float32)

_NEDGE = _BINS - 1
_NACC = 2 * _NEDGE + 1


def _splat(v, dtype=jnp.float32):
    return jnp.full((16,), v, dtype=dtype)


def _sc_partials(d_arr, target, start, count):
    info = plsc.get_sparse_core_info()
    ncores, nsub = info.num_cores, info.num_subcores
    nworkers = ncores * nsub
    assert count % (nworkers * 16) == 0
    per_worker = count // nworkers
    chunk = 8192 if per_worker % 8192 == 0 else per_worker
    nchunks = per_worker // chunk
    nvec = chunk // 16

    mesh = plsc.VectorSubcoreMesh(core_axis_name="c", subcore_axis_name="s")

    @functools.partial(
        pl.kernel,
        mesh=mesh,
        compiler_params=pltpu.CompilerParams(needs_layout_passes=False),
        out_type=jax.ShapeDtypeStruct((nworkers, _NACC, 16), jnp.float32),
        scratch_types=[
            pltpu.VMEM((chunk,), jnp.float32),
            pltpu.VMEM((chunk,), jnp.float32),
            pltpu.VMEM((chunk,), jnp.int32),
            pltpu.VMEM((chunk,), jnp.int32),
            pltpu.VMEM((_NACC, 16), jnp.float32),
            pltpu.SemaphoreType.DMA,
            pltpu.SemaphoreType.DMA,
            pltpu.SemaphoreType.DMA,
            pltpu.SemaphoreType.DMA,
        ],
    )
    def sc_kernel(in_hbm, tgt_hbm, out_hbm, in_buf0, in_buf1, tgt_buf0,
                  tgt_buf1, acc_v, sem_i0, sem_i1, sem_t0, sem_t1):
        in_bufs = (in_buf0, in_buf1)
        tgt_bufs = (tgt_buf0, tgt_buf1)
        wid = lax.axis_index("s") * ncores + lax.axis_index("c")
        base = start + wid * per_worker
        sems_i = (sem_i0, sem_i1)
        sems_t = (sem_t0, sem_t1)

        def launch(c):
            slot = c % 2
            off = base + c * chunk
            cp_i = pltpu.async_copy(
                in_hbm.at[pl.ds(off, chunk)], in_bufs[slot], sems_i[slot])
            cp_t = pltpu.async_copy(
                tgt_hbm.at[pl.ds(off, chunk)], tgt_bufs[slot], sems_t[slot])
            return cp_i, cp_t

        ones_i = _splat(1, jnp.int32)
        zero = _splat(0.0)
        pcoef = [_splat(v) for v in (
            2.554673e-07, 0.9999671, -0.49928504, 0.32722571, -0.22316587,
            0.13083343, -0.052437536, 0.01000929)]
        taus = [_splat(float(t)) for t in _TAUS]

        zero_cnt = _splat(0, jnp.int32)
        accs = [zero_cnt] * _NEDGE + [zero] * (_NEDGE + 1)

        pending = launch(0)
        for c in range(nchunks):
            nxt = launch(c + 1) if c + 1 < nchunks else None
            pending[0].wait()
            pending[1].wait()
            slot = c % 2
            in_view = in_bufs[slot]
            tgt_view = tgt_bufs[slot]

            def body(v, carry):
                d = in_view[pl.ds(v * 16, 16)]
                t = tgt_view[pl.ds(v * 16, 16)]
                u = jnp.where(t == ones_i, d, -d)
                e = jnp.exp(-jnp.abs(d))
                log1pe = pcoef[0] + e * (pcoef[1] + e * (pcoef[2] + e * (
                    pcoef[3] + e * (pcoef[4] + e * (pcoef[5] + e * (
                        pcoef[6] + e * pcoef[7]))))))
                ce = jnp.maximum(u, zero) + log1pe
                out = list(carry)
                out[2 * _NEDGE] = out[2 * _NEDGE] + ce
                for i in range(_NEDGE):
                    m = d >= taus[i]
                    out[i] = out[i] + plsc.all_reduce_population_count(m)
                    out[_NEDGE + i] = out[_NEDGE + i] + jnp.where(m, ce, zero)
                return tuple(out)

            accs = list(lax.fori_loop(0, nvec, body, tuple(accs)))
            pending = nxt

        for i in range(_NACC):
            acc_v[i] = accs[i].astype(jnp.float32)
        pltpu.sync_copy(acc_v, out_hbm.at[wid])

    return sc_kernel(d_arr, target)


def _tc_hist_body(d_ref, t_ref, out_ref):
    @pl.when(pl.program_id(0) == 0)
    def _():
        out_ref[...] = jnp.zeros_like(out_ref)

    d = d_ref[...]
    t = t_ref[...]
    u = jnp.where(t == 1, d, -d)
    e = jnp.exp(-jnp.abs(d))
    ce = jnp.maximum(u, 0.0) + jnp.log1p(e)
    acc = out_ref[...]
    rows = [None] * _NACC
    rows[2 * _NEDGE] = jnp.sum(ce, axis=0)
    for i in range(_NEDGE):
        m = d >= _TAUS[i]
        rows[i] = jnp.sum(jnp.where(m, 1.0, 0.0), axis=0)
        rows[_NEDGE + i] = jnp.sum(jnp.where(m, ce, 0.0), axis=0)
    out_ref[...] = acc + jnp.stack(rows, axis=0)


def _tc_partials(d2, t2):
    rows = d2.shape[0]
    blk = 1024
    assert rows % blk == 0
    return pl.pallas_call(
        _tc_hist_body,
        grid=(rows // blk,),
        in_specs=[
            pl.BlockSpec((blk, 128), lambda i: (i, 0)),
            pl.BlockSpec((blk, 128), lambda i: (i, 0)),
        ],
        out_specs=pl.BlockSpec((_NACC, 128), lambda i: (0, 0)),
        out_shape=jax.ShapeDtypeStruct((_NACC, 128), jnp.float32),
        compiler_params=pltpu.CompilerParams(
            dimension_semantics=("arbitrary",)),
    )(d2, t2)


def _finalize_body(n, sc_ref, tc_ref, out_ref):
    x = sc_ref[...]
    s2 = jnp.sum(x, axis=0)
    sc_rows = jnp.sum(s2, axis=1)
    tc_rows = jnp.sum(tc_ref[...], axis=1)
    cnt_cum = (sc_rows[0:_NEDGE] * np.float32(1.0 / 16.0)
               + tc_rows[0:_NEDGE])
    ce_cum = sc_rows[_NEDGE:2 * _NEDGE] + tc_rows[_NEDGE:2 * _NEDGE]
    ce_tot = sc_rows[2 * _NEDGE] + tc_rows[2 * _NEDGE]
    n_f = jnp.full((1,), float(n), jnp.float32)
    zero1 = jnp.zeros((1,), jnp.float32)
    s_lo = jnp.concatenate([n_f, cnt_cum])
    s_hi = jnp.concatenate([cnt_cum, zero1])
    ce_lo = jnp.concatenate([jnp.reshape(ce_tot, (1,)), ce_cum])
    ce_hi = jnp.concatenate([ce_cum, zero1])
    cnt_b = s_lo - s_hi
    ce_b = ce_lo - ce_hi
    per_bin = jnp.where(cnt_b > 0.5, ce_b / jnp.maximum(cnt_b, 1.0), 0.0)
    loss = jnp.sum(per_bin) * np.float32(n / _BINS)
    out_ref[...] = jnp.reshape(loss, (1, 1))


_SC_SHARE_NUM, _SC_SHARE_DEN = 3, 8


def kernel(inputs, target):
    n = inputs.shape[0]
    target = target.astype(jnp.int32)
    d_arr = inputs[:, 0] - inputs[:, 1]
    n_sc = (n * _SC_SHARE_NUM // _SC_SHARE_DEN) // 262144 * 262144
    n_tc = n - n_sc
    part_sc = _sc_partials(d_arr, target, n_tc, n_sc)
    part_tc = _tc_partials(d_arr[:n_tc].reshape(-1, 128),
                           target[:n_tc].reshape(-1, 128))
    loss = pl.pallas_call(
        functools.partial(_finalize_body, n),
        out_shape=jax.ShapeDtypeStruct((1, 1), jnp.float32),
    )(part_sc, part_tc)
    return jnp.reshape(loss, ())
